# baseline (speedup 1.0000x reference)
import functools

import jax
import jax.numpy as jnp
from jax import lax
from jax.experimental import pallas as pl
from jax.experimental.pallas import tpu as pltpu
from jax.experimental.pallas import tpu_sc as plsc

B, T, W = 1024, 200, 10
N = B * T
D = 128
NC, NS = 2, 16
NW = NC * NS
PER_TILE = N // NW
C = 32
CHUNKS = PER_TILE // C
IDXB = W * C
D2 = D // 2


def _sc_gather(table_f, table_l, table_boc, idx_all):
    mesh = plsc.VectorSubcoreMesh(core_axis_name="c", subcore_axis_name="s")

    scratch = [
        [pltpu.VMEM((IDXB,), jnp.int32) for _ in range(2)],
        [pltpu.VMEM((C, D2), jnp.int32) for _ in range(2)],
        [pltpu.VMEM((C, D2), jnp.int32) for _ in range(2)],
        [[pltpu.VMEM((C, D2), jnp.int32) for _ in range(8)] for _ in range(2)],
        [pltpu.VMEM((C, D2), jnp.int32) for _ in range(2)],
        [pltpu.SemaphoreType.DMA for _ in range(2)],
        [pltpu.SemaphoreType.DMA for _ in range(2)],
        [pltpu.SemaphoreType.DMA for _ in range(2)],
    ]

    @functools.partial(
        pl.kernel,
        out_type=[jax.ShapeDtypeStruct((N, D2), jnp.int32)] * 3,
        mesh=mesh,
        scratch_types=scratch,
        compiler_params=pltpu.CompilerParams(use_tc_tiling_on_sc=False),
    )
    def sc_kernel(tf_h, tl_h, tb_h, idx_h, ef_h, eb_h, el_h,
                  idxv, buf_f, buf_l, bocs, ebuf, sem_i, sem_g, sem_w):
        wid = lax.axis_index("s") * NC + lax.axis_index("c")
        blk0 = wid * CHUNKS

        def idx_copy(s, k):
            return pltpu.make_async_copy(idx_h.at[blk0 + k], idxv[s], sem_i[s])

        def gather_descs(s):
            ds_ = [pltpu.make_async_copy(tf_h.at[idxv[s].at[pl.ds(0, C)]],
                                         buf_f[s], sem_g[s]),
                   pltpu.make_async_copy(tl_h.at[idxv[s].at[pl.ds(C, C)]],
                                         buf_l[s], sem_g[s])]
            for j in range(8):
                ds_.append(pltpu.make_async_copy(
                    tb_h.at[idxv[s].at[pl.ds((2 + j) * C, C)]],
                    bocs[s][j], sem_g[s]))
            return ds_

        def write_descs(s, k):
            base = wid * PER_TILE + k * C
            return [pltpu.make_async_copy(buf_f[s], ef_h.at[pl.ds(base, C)], sem_w[s]),
                    pltpu.make_async_copy(ebuf[s], eb_h.at[pl.ds(base, C)], sem_w[s]),
                    pltpu.make_async_copy(buf_l[s], el_h.at[pl.ds(base, C)], sem_w[s])]

        idx_copy(0, 0).start()
        idx_copy(0, 0).wait()
        for d_ in gather_descs(0):
            d_.start()
        idx_copy(1, 1).start()

        def outer(i, carry):
            for b_ in range(2):
                k = 2 * i + b_
                s, s1 = b_, 1 - b_

                @pl.when(k >= 1)
                def _():
                    for d_ in write_descs(s1, 0):
                        d_.wait()

                @pl.when(k < CHUNKS - 1)
                def _():
                    idx_copy(s1, 0).wait()
                    for d_ in gather_descs(s1):
                        d_.start()

                for d_ in gather_descs(s):
                    d_.wait()

                @pl.when(k < CHUNKS - 2)
                def _():
                    idx_copy(s, k + 2).start()

                def tok(t, tc):
                    for c in range(D2 // 16):
                        sl = pl.ds(c * 16, 16)
                        acc_lo = acc_hi = None
                        for j in range(8):
                            w = bocs[s][j][t, sl]
                            lo = lax.bitcast_convert_type(
                                lax.shift_left(w, 16), jnp.float32)
                            hi = lax.bitcast_convert_type(
                                lax.bitwise_and(w, jnp.int32(-65536)),
                                jnp.float32)
                            acc_lo = lo if j == 0 else acc_lo + lo
                            acc_hi = hi if j == 0 else acc_hi + hi
                        yl = lax.bitcast_convert_type(acc_lo, jnp.int32)
                        yh = lax.bitcast_convert_type(acc_hi, jnp.int32)
                        rl = yl + jnp.int32(0x7FFF) + lax.bitwise_and(
                            lax.shift_right_logical(yl, 16), jnp.int32(1))
                        rh = yh + jnp.int32(0x7FFF) + lax.bitwise_and(
                            lax.shift_right_logical(yh, 16), jnp.int32(1))
                        ebuf[s][t, sl] = lax.bitwise_or(
                            lax.bitwise_and(rh, jnp.int32(-65536)),
                            lax.shift_right_logical(rl, 16))
                    return tc

                lax.fori_loop(0, C, tok, 0)
                for d_ in write_descs(s, k):
                    d_.start()
            return carry

        lax.fori_loop(0, CHUNKS // 2, outer, 0)
        for d_ in write_descs((CHUNKS - 1) % 2, 0):
            d_.wait()

    return sc_kernel(table_f, table_l, table_boc, idx_all)


R = 1024


def _mlp_body(ef, eb, el, w1, b1, w2, b2, out):
    x = jnp.concatenate([ef[...], eb[...], el[...]], axis=1)
    h = jnp.maximum(jnp.dot(x, w1[...], preferred_element_type=jnp.float32)
                    + b1[...], 0.0)
    y = jnp.maximum(jnp.dot(h.astype(jnp.bfloat16), w2[...],
                            preferred_element_type=jnp.float32)
                    + b2[...], 0.0)
    out[...] = y


def _tc_mlp(ef, eb, el, w1, b1, w2, b2):
    grid = (N // R,)
    row_spec = pl.BlockSpec((R, D), lambda i: (i, 0))
    full = lambda shape: pl.BlockSpec(shape, lambda i: (0, 0))
    return pl.pallas_call(
        _mlp_body,
        grid=grid,
        in_specs=[row_spec, row_spec, row_spec,
                  full((3 * D, 3 * D)), full((1, 3 * D)),
                  full((3 * D, D)), full((1, D))],
        out_specs=row_spec,
        out_shape=jax.ShapeDtypeStruct((N, D), jnp.float32),
    )(ef, eb, el, w1, b1, w2, b2)


def kernel(src_tokens, table_boc, table_f, table_l, W1, b1, W2, b2):
    flat = src_tokens.reshape(N, W).astype(jnp.int32)
    idx_all = (flat.reshape(NW, CHUNKS, C, W)
               .transpose(0, 1, 3, 2)
               .reshape(NW * CHUNKS, W * C))
    def to_i32(t):
        return lax.bitcast_convert_type(
            t.astype(jnp.bfloat16).reshape(-1, D2, 2), jnp.int32)

    ef, eb, el = _sc_gather(to_i32(table_f), to_i32(table_l),
                            to_i32(table_boc), idx_all)

    def to_bf16(e):
        return lax.bitcast_convert_type(e, jnp.bfloat16).reshape(N, D)

    ef, eb, el = to_bf16(ef), to_bf16(eb), to_bf16(el)
    w1 = jnp.concatenate([W1[:D], W1[D:2 * D] * (1.0 / 8.0), W1[2 * D:]],
                         axis=0).astype(jnp.bfloat16)
    out = _tc_mlp(ef, eb, el, w1, b1.reshape(1, -1),
                  W2.astype(jnp.bfloat16), b2.reshape(1, -1))
    return out.reshape(B, T, D)


# speedup vs baseline: 1.6130x; 1.6130x over previous
import functools

import jax
import jax.numpy as jnp
from jax import lax
from jax.experimental import pallas as pl
from jax.experimental.pallas import tpu as pltpu
from jax.experimental.pallas import tpu_sc as plsc

B, T, W = 1024, 200, 10
N = B * T
D = 128
D2 = D // 2
V = 100000
V2 = V // 2
NC, NS = 2, 16
NW = NC * NS
PER_TILE = N // NW
C = 64
CHUNKS = PER_TILE // C
IDXR = W * C // 128

_I16_MASK = -65536


def _rne_lo(y):
    r = y + jnp.int32(0x7FFF) + lax.bitwise_and(
        lax.shift_right_logical(y, 16), jnp.int32(1))
    return lax.shift_right_logical(r, 16)


def _rne_hi(y):
    r = y + jnp.int32(0x7FFF) + lax.bitwise_and(
        lax.shift_right_logical(y, 16), jnp.int32(1))
    return lax.bitwise_and(r, jnp.int32(_I16_MASK))


RP = 2000


def _pack_body(x1, x2, out):
    def pk(x):
        y = lax.bitcast_convert_type(x[...], jnp.int32)
        return lax.bitwise_or(_rne_lo(y[:, :D2]), _rne_hi(y[:, D2:]))

    out[...] = jnp.concatenate([pk(x1), pk(x2)], axis=1)


def _pack_table(t):
    nblk = V2 // RP
    return pl.pallas_call(
        _pack_body,
        grid=(nblk,),
        in_specs=[pl.BlockSpec((RP, D), lambda i: (i, 0)),
                  pl.BlockSpec((RP, D), lambda i, _n=nblk: (i + _n, 0))],
        out_specs=pl.BlockSpec((RP, D), lambda i: (i, 0)),
        out_shape=jax.ShapeDtypeStruct((V2, D), jnp.int32),
    )(t, t).reshape(V, D2)


def _sc_gather(table_f, table_l, table_boc, idx_all):
    mesh = plsc.VectorSubcoreMesh(core_axis_name="c", subcore_axis_name="s")

    scratch = [
        [pltpu.VMEM((IDXR, 128), jnp.int32) for _ in range(2)],
        [pltpu.VMEM((C, D2), jnp.int32) for _ in range(2)],
        [pltpu.VMEM((C, D2), jnp.int32) for _ in range(2)],
        [[pltpu.VMEM((C, D2), jnp.int32) for _ in range(8)] for _ in range(2)],
        [pltpu.VMEM((C, D2), jnp.int32) for _ in range(2)],
        [pltpu.SemaphoreType.DMA for _ in range(2)],
        [pltpu.SemaphoreType.DMA for _ in range(2)],
        [pltpu.SemaphoreType.DMA for _ in range(2)],
    ]

    @functools.partial(
        pl.kernel,
        out_type=[jax.ShapeDtypeStruct((N, D2), jnp.int32)] * 3,
        mesh=mesh,
        scratch_types=scratch,
        compiler_params=pltpu.CompilerParams(use_tc_tiling_on_sc=False),
    )
    def sc_kernel(tf_h, tl_h, tb_h, idx_h, ef_h, eb_h, el_h,
                  idxv, buf_f, buf_l, bocs, ebuf, sem_i, sem_g, sem_w):
        wid = lax.axis_index("s") * NC + lax.axis_index("c")
        blk0 = wid * CHUNKS

        def idx_copy(s, k):
            return pltpu.make_async_copy(
                idx_h.at[pl.ds((blk0 + k) * IDXR, IDXR)], idxv[s], sem_i[s])

        def gather_descs(s):
            def idx_ref(field):
                return idxv[s].at[field // 2, pl.ds((field % 2) * C, C)]

            ds_ = [pltpu.make_async_copy(tf_h.at[idx_ref(0)], buf_f[s], sem_g[s]),
                   pltpu.make_async_copy(tl_h.at[idx_ref(1)], buf_l[s], sem_g[s])]
            for j in range(8):
                ds_.append(pltpu.make_async_copy(
                    tb_h.at[idx_ref(2 + j)], bocs[s][j], sem_g[s]))
            return ds_

        def write_descs(s, k):
            base = wid * PER_TILE + k * C
            return [pltpu.make_async_copy(buf_f[s], ef_h.at[pl.ds(base, C)], sem_w[s]),
                    pltpu.make_async_copy(ebuf[s], eb_h.at[pl.ds(base, C)], sem_w[s]),
                    pltpu.make_async_copy(buf_l[s], el_h.at[pl.ds(base, C)], sem_w[s])]

        idx_copy(0, 0).start()
        idx_copy(0, 0).wait()
        for d_ in gather_descs(0):
            d_.start()
        idx_copy(1, 1).start()

        def outer(i, carry):
            for b_ in range(2):
                k = 2 * i + b_
                s, s1 = b_, 1 - b_

                @pl.when(k >= 1)
                def _():
                    for d_ in write_descs(s1, 0):
                        d_.wait()

                @pl.when(k < CHUNKS - 1)
                def _():
                    idx_copy(s1, 0).wait()
                    for d_ in gather_descs(s1):
                        d_.start()

                for d_ in gather_descs(s):
                    d_.wait()

                @pl.when(k < CHUNKS - 2)
                def _():
                    idx_copy(s, k + 2).start()

                def tok(t, tc):
                    for c in range(D2 // 16):
                        sl = pl.ds(c * 16, 16)
                        acc_lo = acc_hi = None
                        for j in range(8):
                            w = bocs[s][j][t, sl]
                            lo = lax.bitcast_convert_type(
                                lax.shift_left(w, 16), jnp.float32)
                            hi = lax.bitcast_convert_type(
                                lax.bitwise_and(w, jnp.int32(_I16_MASK)),
                                jnp.float32)
                            acc_lo = lo if j == 0 else acc_lo + lo
                            acc_hi = hi if j == 0 else acc_hi + hi
                        yl = lax.bitcast_convert_type(acc_lo, jnp.int32)
                        yh = lax.bitcast_convert_type(acc_hi, jnp.int32)
                        ebuf[s][t, sl] = lax.bitwise_or(_rne_lo(yl),
                                                        _rne_hi(yh))
                    return tc

                lax.fori_loop(0, C, tok, 0)
                for d_ in write_descs(s, k):
                    d_.start()
            return carry

        lax.fori_loop(0, CHUNKS // 2, outer, 0)
        for d_ in write_descs((CHUNKS - 1) % 2, 0):
            d_.wait()

    return sc_kernel(table_f, table_l, table_boc, idx_all)


R = 1024


def _mlp_body(ef, eb, el, w1, b1, w2, b2, out):
    x = jnp.concatenate([ef[...], eb[...], el[...]], axis=1)
    h = jnp.maximum(jnp.dot(x, w1[...], preferred_element_type=jnp.float32)
                    + b1[...], 0.0)
    y = jnp.maximum(jnp.dot(h.astype(jnp.bfloat16), w2[...],
                            preferred_element_type=jnp.float32)
                    + b2[...], 0.0)
    out[...] = y


def _tc_mlp(ef, eb, el, w1, b1, w2, b2):
    grid = (N // R,)
    row_spec = pl.BlockSpec((R, D), lambda i: (i, 0))
    full = lambda shape: pl.BlockSpec(shape, lambda i: (0, 0))
    return pl.pallas_call(
        _mlp_body,
        grid=grid,
        in_specs=[row_spec, row_spec, row_spec,
                  full((3 * D, 3 * D)), full((1, 3 * D)),
                  full((3 * D, D)), full((1, D))],
        out_specs=row_spec,
        out_shape=jax.ShapeDtypeStruct((N, D), jnp.float32),
    )(ef, eb, el, w1, b1, w2, b2)


def kernel(src_tokens, table_boc, table_f, table_l, W1, b1, W2, b2):
    flat = src_tokens.reshape(N, W).astype(jnp.int32)
    flat = jnp.where(flat < V2, 2 * flat, 2 * (flat - V2) + 1)
    idx_all = (flat.reshape(NW, CHUNKS, C, W)
               .transpose(0, 1, 3, 2)
               .reshape(NW * CHUNKS * IDXR, 128))

    ef, eb, el = _sc_gather(_pack_table(table_f), _pack_table(table_l),
                            _pack_table(table_boc), idx_all)

    def to_bf16(e):
        return lax.bitcast_convert_type(e, jnp.bfloat16).reshape(N, D)

    ef, eb, el = to_bf16(ef), to_bf16(eb), to_bf16(el)

    p = (jnp.arange(D) // 2) + (jnp.arange(D) % 2) * D2
    w1 = jnp.concatenate([W1[:D][p], (W1[D:2 * D] * (1.0 / 8.0))[p],
                          W1[2 * D:][p]], axis=0).astype(jnp.bfloat16)
    out = _tc_mlp(ef, eb, el, w1, b1.reshape(1, -1),
                  W2.astype(jnp.bfloat16), b2.reshape(1, -1))
    return out.reshape(B, T, D)


# speedup vs baseline: 4.2644x; 2.6438x over previous
import functools

import jax
import jax.numpy as jnp
from jax import lax
from jax.experimental import pallas as pl
from jax.experimental.pallas import tpu as pltpu
from jax.experimental.pallas import tpu_sc as plsc

B, T, W = 1024, 200, 10
N = B * T
D = 128
NC, NS = 2, 16
NW = NC * NS
PER_TILE = N // NW
C = 40
CHUNKS = PER_TILE // C
IDXB = W * C


def _sc_gather(table_f, table_l, table_boc, idx_all):
    mesh = plsc.VectorSubcoreMesh(core_axis_name="c", subcore_axis_name="s")

    scratch = [
        [pltpu.VMEM((IDXB,), jnp.int32) for _ in range(2)],
        [pltpu.VMEM((C, D), jnp.float32) for _ in range(2)],
        [pltpu.VMEM((C, D), jnp.float32) for _ in range(2)],
        [[pltpu.VMEM((C, D), jnp.float32) for _ in range(8)] for _ in range(2)],
        [pltpu.VMEM((C, D), jnp.float32) for _ in range(2)],
        [pltpu.SemaphoreType.DMA for _ in range(2)],
        [pltpu.SemaphoreType.DMA for _ in range(2)],
        [pltpu.SemaphoreType.DMA for _ in range(2)],
    ]

    @functools.partial(
        pl.kernel,
        out_type=[jax.ShapeDtypeStruct((N, D), jnp.float32)] * 3,
        mesh=mesh,
        scratch_types=scratch,
    )
    def sc_kernel(tf_h, tl_h, tb_h, idx_h, ef_h, eb_h, el_h,
                  idxv, buf_f, buf_l, bocs, ebuf, sem_i, sem_g, sem_w):
        wid = lax.axis_index("s") * NC + lax.axis_index("c")
        blk0 = wid * CHUNKS

        def idx_copy(s, k):
            return pltpu.make_async_copy(idx_h.at[blk0 + k], idxv[s], sem_i[s])

        def gather_descs(s):
            ds_ = [pltpu.make_async_copy(tf_h.at[idxv[s].at[pl.ds(0, C)]],
                                         buf_f[s], sem_g[s]),
                   pltpu.make_async_copy(tl_h.at[idxv[s].at[pl.ds(C, C)]],
                                         buf_l[s], sem_g[s])]
            for j in range(8):
                ds_.append(pltpu.make_async_copy(
                    tb_h.at[idxv[s].at[pl.ds((2 + j) * C, C)]],
                    bocs[s][j], sem_g[s]))
            return ds_

        def write_descs(s, k):
            base = wid * PER_TILE + k * C
            return [pltpu.make_async_copy(buf_f[s], ef_h.at[pl.ds(base, C)], sem_w[s]),
                    pltpu.make_async_copy(ebuf[s], eb_h.at[pl.ds(base, C)], sem_w[s]),
                    pltpu.make_async_copy(buf_l[s], el_h.at[pl.ds(base, C)], sem_w[s])]

        idx_copy(0, 0).start()
        idx_copy(0, 0).wait()
        for d_ in gather_descs(0):
            d_.start()
        idx_copy(1, 1).start()

        def outer(i, carry):
            for b_ in range(2):
                k = 2 * i + b_
                s, s1 = b_, 1 - b_

                @pl.when(k >= 1)
                def _():
                    for d_ in write_descs(s1, 0):
                        d_.wait()

                @pl.when(k < CHUNKS - 1)
                def _():
                    idx_copy(s1, 0).wait()
                    for d_ in gather_descs(s1):
                        d_.start()

                for d_ in gather_descs(s):
                    d_.wait()

                @pl.when(k < CHUNKS - 2)
                def _():
                    idx_copy(s, k + 2).start()

                def tok(t, tc):
                    for c in range(D // 16):
                        sl = pl.ds(c * 16, 16)
                        v = bocs[s][0][t, sl]
                        for j in range(1, 8):
                            v = v + bocs[s][j][t, sl]
                        ebuf[s][t, sl] = v
                    return tc

                lax.fori_loop(0, C, tok, 0)
                for d_ in write_descs(s, k):
                    d_.start()
            return carry

        lax.fori_loop(0, CHUNKS // 2, outer, 0)
        for d_ in write_descs((CHUNKS - 1) % 2, 0):
            d_.wait()

    return sc_kernel(table_f, table_l, table_boc, idx_all)


R = 1024


def _mlp_body(ef, eb, el, w1, b1, w2, b2, out):
    x = jnp.concatenate([ef[...], eb[...], el[...]], axis=1).astype(jnp.bfloat16)
    h = jnp.maximum(jnp.dot(x, w1[...], preferred_element_type=jnp.float32)
                    + b1[...], 0.0)
    y = jnp.maximum(jnp.dot(h.astype(jnp.bfloat16), w2[...],
                            preferred_element_type=jnp.float32)
                    + b2[...], 0.0)
    out[...] = y


def _tc_mlp(ef, eb, el, w1, b1, w2, b2):
    grid = (N // R,)
    row_spec = pl.BlockSpec((R, D), lambda i: (i, 0))
    full = lambda shape: pl.BlockSpec(shape, lambda i: (0, 0))
    return pl.pallas_call(
        _mlp_body,
        grid=grid,
        in_specs=[row_spec, row_spec, row_spec,
                  full((3 * D, 3 * D)), full((1, 3 * D)),
                  full((3 * D, D)), full((1, D))],
        out_specs=row_spec,
        out_shape=jax.ShapeDtypeStruct((N, D), jnp.float32),
    )(ef, eb, el, w1, b1, w2, b2)


def kernel(src_tokens, table_boc, table_f, table_l, W1, b1, W2, b2):
    flat = src_tokens.reshape(N, W).astype(jnp.int32)
    idx_all = (flat.reshape(NW, CHUNKS, C, W)
               .transpose(0, 1, 3, 2)
               .reshape(NW * CHUNKS, W * C))
    ef, eb, el = _sc_gather(table_f, table_l, table_boc, idx_all)
    w1 = jnp.concatenate([W1[:D], W1[D:2 * D] * (1.0 / 8.0), W1[2 * D:]],
                         axis=0).astype(jnp.bfloat16)
    out = _tc_mlp(ef, eb, el, w1, b1.reshape(1, -1),
                  W2.astype(jnp.bfloat16), b2.reshape(1, -1))
    return out.reshape(B, T, D)


# speedup vs baseline: 4.2995x; 1.0082x over previous
import functools

import jax
import jax.numpy as jnp
from jax import lax
from jax.experimental import pallas as pl
from jax.experimental.pallas import tpu as pltpu
from jax.experimental.pallas import tpu_sc as plsc

B, T, W = 1024, 200, 10
N = B * T
D = 128
NC, NS = 2, 16
NW = NC * NS
PER_TILE = N // NW
C = 40
CHUNKS = PER_TILE // C
IDXB = W * C


def _sc_gather(table_f, table_l, table_boc, idx_all, n_rows, chunks):
    per_tile = n_rows // NW
    mesh = plsc.VectorSubcoreMesh(core_axis_name="c", subcore_axis_name="s")

    scratch = [
        [pltpu.VMEM((IDXB,), jnp.int32) for _ in range(2)],
        [pltpu.VMEM((C, D), jnp.float32) for _ in range(2)],
        [pltpu.VMEM((C, D), jnp.float32) for _ in range(2)],
        [[pltpu.VMEM((C, D), jnp.float32) for _ in range(8)] for _ in range(2)],
        [pltpu.VMEM((C, D), jnp.float32) for _ in range(2)],
        [pltpu.SemaphoreType.DMA for _ in range(2)],
        [pltpu.SemaphoreType.DMA for _ in range(2)],
        [pltpu.SemaphoreType.DMA for _ in range(2)],
    ]

    @functools.partial(
        pl.kernel,
        out_type=[jax.ShapeDtypeStruct((n_rows, D), jnp.float32)] * 3,
        mesh=mesh,
        scratch_types=scratch,
    )
    def sc_kernel(tf_h, tl_h, tb_h, idx_h, ef_h, eb_h, el_h,
                  idxv, buf_f, buf_l, bocs, ebuf, sem_i, sem_g, sem_w):
        wid = lax.axis_index("s") * NC + lax.axis_index("c")
        blk0 = wid * chunks

        def idx_copy(s, k):
            return pltpu.make_async_copy(idx_h.at[blk0 + k], idxv[s], sem_i[s])

        def gather_descs(s):
            ds_ = [pltpu.make_async_copy(tf_h.at[idxv[s].at[pl.ds(0, C)]],
                                         buf_f[s], sem_g[s]),
                   pltpu.make_async_copy(tl_h.at[idxv[s].at[pl.ds(C, C)]],
                                         buf_l[s], sem_g[s])]
            for j in range(8):
                ds_.append(pltpu.make_async_copy(
                    tb_h.at[idxv[s].at[pl.ds((2 + j) * C, C)]],
                    bocs[s][j], sem_g[s]))
            return ds_

        def write_descs(s, k):
            base = wid * per_tile + k * C
            return [pltpu.make_async_copy(buf_f[s], ef_h.at[pl.ds(base, C)], sem_w[s]),
                    pltpu.make_async_copy(ebuf[s], eb_h.at[pl.ds(base, C)], sem_w[s]),
                    pltpu.make_async_copy(buf_l[s], el_h.at[pl.ds(base, C)], sem_w[s])]

        idx_copy(0, 0).start()
        idx_copy(0, 0).wait()
        for d_ in gather_descs(0):
            d_.start()
        idx_copy(1, 1).start()

        def outer(i, carry):
            for b_ in range(2):
                k = 2 * i + b_
                s, s1 = b_, 1 - b_

                @pl.when(k >= 1)
                def _():
                    for d_ in write_descs(s1, 0):
                        d_.wait()

                @pl.when(k < chunks - 1)
                def _():
                    idx_copy(s1, 0).wait()
                    for d_ in gather_descs(s1):
                        d_.start()

                for d_ in gather_descs(s):
                    d_.wait()

                @pl.when(k < chunks - 2)
                def _():
                    idx_copy(s, k + 2).start()

                def tok(t, tc):
                    for c in range(D // 16):
                        sl = pl.ds(c * 16, 16)
                        v = bocs[s][0][t, sl]
                        for j in range(1, 8):
                            v = v + bocs[s][j][t, sl]
                        ebuf[s][t, sl] = v
                    return tc

                lax.fori_loop(0, C, tok, 0)
                for d_ in write_descs(s, k):
                    d_.start()
            return carry

        lax.fori_loop(0, chunks // 2, outer, 0)
        for d_ in write_descs((chunks - 1) % 2, 0):
            d_.wait()

    return sc_kernel(table_f, table_l, table_boc, idx_all)


R = 1024


def _mlp_body(ef, eb, el, w1, b1, w2, b2, out):
    x = jnp.concatenate([ef[...], eb[...], el[...]], axis=1).astype(jnp.bfloat16)
    h = jnp.maximum(jnp.dot(x, w1[...], preferred_element_type=jnp.float32)
                    + b1[...], 0.0)
    y = jnp.maximum(jnp.dot(h.astype(jnp.bfloat16), w2[...],
                            preferred_element_type=jnp.float32)
                    + b2[...], 0.0)
    out[...] = y


def _tc_mlp(ef, eb, el, w1, b1, w2, b2, n_rows):
    grid = (n_rows // R,)
    row_spec = pl.BlockSpec((R, D), lambda i: (i, 0))
    full = lambda shape: pl.BlockSpec(shape, lambda i: (0, 0))
    return pl.pallas_call(
        _mlp_body,
        grid=grid,
        in_specs=[row_spec, row_spec, row_spec,
                  full((3 * D, 3 * D)), full((1, 3 * D)),
                  full((3 * D, D)), full((1, D))],
        out_specs=row_spec,
        out_shape=jax.ShapeDtypeStruct((n_rows, D), jnp.float32),
    )(ef, eb, el, w1, b1, w2, b2)


HALVES = 2
N2 = N // HALVES
CHUNKS2 = (N2 // NW) // C


def kernel(src_tokens, table_boc, table_f, table_l, W1, b1, W2, b2):
    flat = src_tokens.reshape(N, W).astype(jnp.int32)
    w1 = jnp.concatenate([W1[:D], W1[D:2 * D] * (1.0 / 8.0), W1[2 * D:]],
                         axis=0).astype(jnp.bfloat16)
    w2 = W2.astype(jnp.bfloat16)
    b1r, b2r = b1.reshape(1, -1), b2.reshape(1, -1)
    outs = []
    for h in range(HALVES):
        fl = flat[h * N2:(h + 1) * N2]
        idx_h = (fl.reshape(NW, CHUNKS2, C, W)
                 .transpose(0, 1, 3, 2)
                 .reshape(NW * CHUNKS2, W * C))
        ef, eb, el = _sc_gather(table_f, table_l, table_boc, idx_h,
                                N2, CHUNKS2)
        outs.append(_tc_mlp(ef, eb, el, w1, b1r, w2, b2r, N2))
    return jnp.concatenate(outs, axis=0).reshape(B, T, D)


# speedup vs baseline: 4.3085x; 1.0021x over previous
import functools

import jax
import jax.numpy as jnp
from jax import lax
from jax.experimental import pallas as pl
from jax.experimental.pallas import tpu as pltpu
from jax.experimental.pallas import tpu_sc as plsc

B, T, W = 1024, 200, 10
N = B * T
D = 128
NC, NS = 2, 16
NW = NC * NS
PER_TILE = N // NW
C = 40
CHUNKS = PER_TILE // C
IDXB = W * C


def _sc_gather(table_f, table_l, table_boc, idx_all, n_rows, chunks):
    per_tile = n_rows // NW
    mesh = plsc.VectorSubcoreMesh(core_axis_name="c", subcore_axis_name="s")

    scratch = [
        [pltpu.VMEM((IDXB,), jnp.int32) for _ in range(2)],
        [pltpu.VMEM((C, D), jnp.float32) for _ in range(2)],
        [pltpu.VMEM((C, D), jnp.float32) for _ in range(2)],
        [[pltpu.VMEM((C, D), jnp.float32) for _ in range(8)] for _ in range(2)],
        [pltpu.VMEM((C, D), jnp.float32) for _ in range(2)],
        [pltpu.SemaphoreType.DMA for _ in range(2)],
        [pltpu.SemaphoreType.DMA for _ in range(2)],
        [pltpu.SemaphoreType.DMA for _ in range(2)],
    ]

    @functools.partial(
        pl.kernel,
        out_type=[jax.ShapeDtypeStruct((n_rows, D), jnp.float32)] * 3,
        mesh=mesh,
        scratch_types=scratch,
    )
    def sc_kernel(tf_h, tl_h, tb_h, idx_h, ef_h, eb_h, el_h,
                  idxv, buf_f, buf_l, bocs, ebuf, sem_i, sem_g, sem_w):
        wid = lax.axis_index("s") * NC + lax.axis_index("c")
        blk0 = wid * chunks

        def idx_copy(s, k):
            return pltpu.make_async_copy(idx_h.at[blk0 + k], idxv[s], sem_i[s])

        def gather_descs(s):
            ds_ = [pltpu.make_async_copy(tf_h.at[idxv[s].at[pl.ds(0, C)]],
                                         buf_f[s], sem_g[s]),
                   pltpu.make_async_copy(tl_h.at[idxv[s].at[pl.ds(C, C)]],
                                         buf_l[s], sem_g[s])]
            for j in range(8):
                ds_.append(pltpu.make_async_copy(
                    tb_h.at[idxv[s].at[pl.ds((2 + j) * C, C)]],
                    bocs[s][j], sem_g[s]))
            return ds_

        def write_descs(s, k):
            base = wid * per_tile + k * C
            return [pltpu.make_async_copy(buf_f[s], ef_h.at[pl.ds(base, C)], sem_w[s]),
                    pltpu.make_async_copy(ebuf[s], eb_h.at[pl.ds(base, C)], sem_w[s]),
                    pltpu.make_async_copy(buf_l[s], el_h.at[pl.ds(base, C)], sem_w[s])]

        idx_copy(0, 0).start()
        idx_copy(0, 0).wait()
        for d_ in gather_descs(0):
            d_.start()
        idx_copy(1, 1).start()

        def outer(i, carry):
            for b_ in range(2):
                k = 2 * i + b_
                s, s1 = b_, 1 - b_

                @pl.when(k >= 1)
                def _():
                    for d_ in write_descs(s1, 0):
                        d_.wait()

                @pl.when(k < chunks - 1)
                def _():
                    idx_copy(s1, 0).wait()
                    for d_ in gather_descs(s1):
                        d_.start()

                for d_ in gather_descs(s):
                    d_.wait()

                @pl.when(k < chunks - 2)
                def _():
                    idx_copy(s, k + 2).start()

                def tok(t, tc):
                    for c in range(D // 16):
                        sl = pl.ds(c * 16, 16)
                        v = bocs[s][0][t, sl]
                        for j in range(1, 8):
                            v = v + bocs[s][j][t, sl]
                        ebuf[s][t, sl] = v
                    return tc

                lax.fori_loop(0, C, tok, 0)
                for d_ in write_descs(s, k):
                    d_.start()
            return carry

        lax.fori_loop(0, chunks // 2, outer, 0)
        for d_ in write_descs((chunks - 1) % 2, 0):
            d_.wait()

    return sc_kernel(table_f, table_l, table_boc, idx_all)


R = 1024


def _mlp_body(ef, eb, el, w1, b1, w2, b2, out):
    x = jnp.concatenate([ef[...], eb[...], el[...]], axis=1).astype(jnp.bfloat16)
    h = jnp.maximum(jnp.dot(x, w1[...], preferred_element_type=jnp.float32)
                    + b1[...], 0.0)
    y = jnp.maximum(jnp.dot(h.astype(jnp.bfloat16), w2[...],
                            preferred_element_type=jnp.float32)
                    + b2[...], 0.0)
    out[...] = y


def _tc_mlp(ef, eb, el, w1, b1, w2, b2, n_rows):
    grid = (n_rows // R,)
    row_spec = pl.BlockSpec((R, D), lambda i: (i, 0))
    full = lambda shape: pl.BlockSpec(shape, lambda i: (0, 0))
    return pl.pallas_call(
        _mlp_body,
        grid=grid,
        in_specs=[row_spec, row_spec, row_spec,
                  full((3 * D, 3 * D)), full((1, 3 * D)),
                  full((3 * D, D)), full((1, D))],
        out_specs=row_spec,
        out_shape=jax.ShapeDtypeStruct((n_rows, D), jnp.float32),
    )(ef, eb, el, w1, b1, w2, b2)


HALVES = 4
N2 = N // HALVES
CHUNKS2 = (N2 // NW) // C


def kernel(src_tokens, table_boc, table_f, table_l, W1, b1, W2, b2):
    flat = src_tokens.reshape(N, W).astype(jnp.int32)
    w1 = jnp.concatenate([W1[:D], W1[D:2 * D] * (1.0 / 8.0), W1[2 * D:]],
                         axis=0).astype(jnp.bfloat16)
    w2 = W2.astype(jnp.bfloat16)
    b1r, b2r = b1.reshape(1, -1), b2.reshape(1, -1)
    outs = []
    for h in range(HALVES):
        fl = flat[h * N2:(h + 1) * N2]
        idx_h = (fl.reshape(NW, CHUNKS2, C, W)
                 .transpose(0, 1, 3, 2)
                 .reshape(NW * CHUNKS2, W * C))
        ef, eb, el = _sc_gather(table_f, table_l, table_boc, idx_h,
                                N2, CHUNKS2)
        outs.append(_tc_mlp(ef, eb, el, w1, b1r, w2, b2r, N2))
    return jnp.concatenate(outs, axis=0).reshape(B, T, D)
